# baseline (speedup 1.0000x reference)
import jax
import jax.numpy as jnp
from jax import lax
from jax.experimental import pallas as pl
from jax.experimental.pallas import tpu as pltpu
from jax.experimental.pallas import tpu_sc as plsc

_N_ATOMS = 100000
_EMB = 128
_INFO = plsc.get_sparse_core_info()
_NW = _INFO.num_cores * _INFO.num_subcores
_CHUNK = 128
_NCH = 26
_ROWS_PER_W = _NCH * _CHUNK
_N_PAD = _NW * _ROWS_PER_W
_NBUF = 4


def _emb_body(z_hbm, w_hbm, out_hbm, idx_v, b0, b1, b2, b3, gsem, ssem):
    bufs = (b0, b1, b2, b3)
    wid = lax.axis_index("s") * _INFO.num_cores + lax.axis_index("c")
    base = wid * _ROWS_PER_W

    pltpu.sync_copy(z_hbm.at[wid], idx_v)
    for j in range(_NCH):
        for i in range(_CHUNK // 16):
            sl = pl.ds(i * 16, 16)
            idx_v[j, sl] = idx_v[j, sl] - 1

    for j in range(_NBUF):
        pltpu.async_copy(w_hbm.at[idx_v.at[j]], bufs[j], gsem)

    for j in range(_NCH):
        b = bufs[j % _NBUF]
        row0 = base + j * _CHUNK
        dst = out_hbm.at[pl.ds(row0, _CHUNK)]
        pltpu.make_async_copy(w_hbm.at[idx_v.at[j]], b, gsem).wait()
        pltpu.async_copy(b, dst, ssem)
        nxt = j + _NBUF
        if nxt < _NCH:
            pltpu.make_async_copy(b, dst, ssem).wait()
            pltpu.async_copy(w_hbm.at[idx_v.at[nxt]], b, gsem)

    for j in range(_NCH - _NBUF, _NCH):
        b = bufs[j % _NBUF]
        row0 = base + j * _CHUNK
        pltpu.make_async_copy(b, out_hbm.at[pl.ds(row0, _CHUNK)], ssem).wait()


@jax.jit
def kernel(Z, W):
    z_pad = jnp.concatenate([Z, jnp.ones((_N_PAD - _N_ATOMS,), jnp.int32)])
    z3 = z_pad.reshape(_NW, _NCH, _CHUNK)
    mesh = plsc.VectorSubcoreMesh(core_axis_name="c", subcore_axis_name="s")
    out = pl.kernel(
        _emb_body,
        out_type=jax.ShapeDtypeStruct((_N_PAD, _EMB), jnp.float32),
        mesh=mesh,
        scratch_types=[
            pltpu.VMEM((_NCH, _CHUNK), jnp.int32),
            pltpu.VMEM((_CHUNK, _EMB), jnp.float32),
            pltpu.VMEM((_CHUNK, _EMB), jnp.float32),
            pltpu.VMEM((_CHUNK, _EMB), jnp.float32),
            pltpu.VMEM((_CHUNK, _EMB), jnp.float32),
            pltpu.SemaphoreType.DMA,
            pltpu.SemaphoreType.DMA,
        ],
    )(z3, W)
    return out[:_N_ATOMS]


# speedup vs baseline: 5.5456x; 5.5456x over previous
import jax
import jax.numpy as jnp
from jax import lax
from jax.experimental import pallas as pl
from jax.experimental.pallas import tpu as pltpu
from jax.experimental.pallas import tpu_sc as plsc

_N_ATOMS = 100000
_EMB = 128
_N_ELEM = 94
_INFO = plsc.get_sparse_core_info()
_NW = _INFO.num_cores * _INFO.num_subcores
_CHUNK = 128
_NCH = 26
_ROWS_PER_W = _NCH * _CHUNK
_N_PAD = _NW * _ROWS_PER_W
_NBUF = 4
_LAG = 2


def _emb_body(z_hbm, w_hbm, out_hbm, idx_v, w_v, bufs, gsem, ssem):
    wid = lax.axis_index("s") * _INFO.num_cores + lax.axis_index("c")
    base = wid * _ROWS_PER_W

    @pl.when(lax.axis_index("s") == 0)
    def _():
        pltpu.sync_copy(w_hbm, w_v)

    plsc.subcore_barrier()
    pltpu.sync_copy(z_hbm.at[wid], idx_v)
    for j in range(_NCH):
        for i in range(_CHUNK // 16):
            sl = pl.ds(i * 16, 16)
            idx_v[j, sl] = idx_v[j, sl] - 1

    def gather(j):
        pltpu.async_copy(w_v.at[idx_v.at[j]], bufs[j % _NBUF], gsem)

    def gather_wait(j):
        pltpu.make_async_copy(w_v.at[idx_v.at[j]], bufs[j % _NBUF], gsem).wait()

    def scatter(j):
        pltpu.async_copy(bufs[j % _NBUF],
                         out_hbm.at[pl.ds(base + j * _CHUNK, _CHUNK)], ssem)

    def scatter_wait(j):
        pltpu.make_async_copy(bufs[j % _NBUF],
                              out_hbm.at[pl.ds(base + j * _CHUNK, _CHUNK)],
                              ssem).wait()

    for j in range(_LAG):
        gather(j)
    for j in range(_NCH):
        gather_wait(j)
        scatter(j)
        if j >= _LAG:
            scatter_wait(j - _LAG)
        if j + _LAG < _NCH:
            gather(j + _LAG)
    for j in range(_NCH - _LAG, _NCH):
        scatter_wait(j)


def _body(z_hbm, w_hbm, out_hbm, idx_v, w_v, b0, b1, b2, b3, gsem, ssem):
    _emb_body(z_hbm, w_hbm, out_hbm, idx_v, w_v, (b0, b1, b2, b3), gsem, ssem)


@jax.jit
def kernel(Z, W):
    z_pad = jnp.concatenate([Z, jnp.ones((_N_PAD - _N_ATOMS,), jnp.int32)])
    z3 = z_pad.reshape(_NW, _NCH, _CHUNK)
    mesh = plsc.VectorSubcoreMesh(core_axis_name="c", subcore_axis_name="s")
    out = pl.kernel(
        _body,
        out_type=jax.ShapeDtypeStruct((_N_PAD, _EMB), jnp.float32),
        mesh=mesh,
        scratch_types=[
            pltpu.VMEM((_NCH, _CHUNK), jnp.int32),
            pltpu.VMEM_SHARED((_N_ELEM, _EMB), jnp.float32),
            pltpu.VMEM((_CHUNK, _EMB), jnp.float32),
            pltpu.VMEM((_CHUNK, _EMB), jnp.float32),
            pltpu.VMEM((_CHUNK, _EMB), jnp.float32),
            pltpu.VMEM((_CHUNK, _EMB), jnp.float32),
            pltpu.SemaphoreType.DMA,
            pltpu.SemaphoreType.DMA,
        ],
    )(z3, W)
    return out[:_N_ATOMS]


# speedup vs baseline: 10.0272x; 1.8081x over previous
import jax
import jax.numpy as jnp
from jax import lax
from jax.experimental import pallas as pl
from jax.experimental.pallas import tpu as pltpu
from jax.experimental.pallas import tpu_sc as plsc

_N_ATOMS = 100000
_EMB = 128
_N_ELEM = 94
_INFO = plsc.get_sparse_core_info()
_NW = _INFO.num_cores * _INFO.num_subcores
_CHUNK = 128
_SLAB = 3128
_SLAB_LAST = _N_ATOMS - (_NW - 1) * _SLAB
_NBUF = 4
_LAG = 2

_SIZES_MAIN = [_CHUNK] * 24 + [56]
_SIZES_LAST = [_CHUNK] * 23 + [88]
_OFFS = [i * _CHUNK for i in range(25)]


def _pipeline(base, sizes, idx_v, w_v, out_hbm, bufs, gsem, ssem):
    nch = len(sizes)

    def gather(j):
        n = sizes[j]
        pltpu.async_copy(w_v.at[idx_v.at[pl.ds(_OFFS[j], n)]],
                         bufs[j % _NBUF].at[pl.ds(0, n)], gsem)

    def gather_wait(j):
        n = sizes[j]
        pltpu.make_async_copy(w_v.at[idx_v.at[pl.ds(_OFFS[j], n)]],
                              bufs[j % _NBUF].at[pl.ds(0, n)], gsem).wait()

    def scatter(j, wait):
        n = sizes[j]
        src = bufs[j % _NBUF].at[pl.ds(0, n)]
        dst = out_hbm.at[pl.ds(pl.multiple_of(base + _OFFS[j], 8), n)]
        if wait:
            pltpu.make_async_copy(src, dst, ssem).wait()
        else:
            pltpu.async_copy(src, dst, ssem)

    for j in range(_LAG):
        gather(j)
    for j in range(nch):
        gather_wait(j)
        scatter(j, wait=False)
        if j >= _LAG:
            scatter(j - _LAG, wait=True)
        if j + _LAG < nch:
            gather(j + _LAG)
    for j in range(nch - _LAG, nch):
        scatter(j, wait=True)


def _body(z_hbm, w_hbm, out_hbm, idx_v, w_v, b0, b1, b2, b3, gsem, ssem):
    bufs = (b0, b1, b2, b3)
    wid = lax.axis_index("s") * _INFO.num_cores + lax.axis_index("c")
    base = pl.multiple_of(wid * _SLAB, 8)

    @pl.when(lax.axis_index("s") == 0)
    def _():
        pltpu.sync_copy(w_hbm, w_v)

    plsc.subcore_barrier()

    @pl.when(wid < _NW - 1)
    def _():
        pltpu.sync_copy(z_hbm.at[pl.ds(base, _SLAB)], idx_v)
        _pipeline(base, _SIZES_MAIN, idx_v, w_v, out_hbm, bufs, gsem, ssem)

    @pl.when(wid == _NW - 1)
    def _():
        pltpu.sync_copy(z_hbm.at[pl.ds(base, _SLAB_LAST)],
                        idx_v.at[pl.ds(0, _SLAB_LAST)])
        _pipeline(base, _SIZES_LAST, idx_v, w_v, out_hbm, bufs, gsem, ssem)


@jax.jit
def kernel(Z, W):
    w95 = jnp.concatenate([jnp.zeros((1, _EMB), jnp.float32), W])
    mesh = plsc.VectorSubcoreMesh(core_axis_name="c", subcore_axis_name="s")
    return pl.kernel(
        _body,
        out_type=jax.ShapeDtypeStruct((_N_ATOMS, _EMB), jnp.float32),
        mesh=mesh,
        scratch_types=[
            pltpu.VMEM((_SLAB,), jnp.int32),
            pltpu.VMEM_SHARED((_N_ELEM + 1, _EMB), jnp.float32),
            pltpu.VMEM((_CHUNK, _EMB), jnp.float32),
            pltpu.VMEM((_CHUNK, _EMB), jnp.float32),
            pltpu.VMEM((_CHUNK, _EMB), jnp.float32),
            pltpu.VMEM((_CHUNK, _EMB), jnp.float32),
            pltpu.SemaphoreType.DMA,
            pltpu.SemaphoreType.DMA,
        ],
    )(Z, w95)


# speedup vs baseline: 10.0419x; 1.0015x over previous
import jax
import jax.numpy as jnp
from jax import lax
from jax.experimental import pallas as pl
from jax.experimental.pallas import tpu as pltpu
from jax.experimental.pallas import tpu_sc as plsc

_N_ATOMS = 100000
_EMB = 128
_N_ELEM = 94
_INFO = plsc.get_sparse_core_info()
_NW = _INFO.num_cores * _INFO.num_subcores
_CHUNK = 128
_SLAB = 3128
_SLAB_LAST = _N_ATOMS - (_NW - 1) * _SLAB
_NBUF = 6
_LAG = 3

_SIZES_MAIN = [_CHUNK] * 24 + [56]
_SIZES_LAST = [_CHUNK] * 23 + [88]
_OFFS = [i * _CHUNK for i in range(25)]


def _pipeline(base, sizes, idx_v, w_v, out_hbm, bufs, gsem, ssem):
    nch = len(sizes)

    def gather(j):
        n = sizes[j]
        pltpu.async_copy(w_v.at[idx_v.at[pl.ds(_OFFS[j], n)]],
                         bufs[j % _NBUF].at[pl.ds(0, n)], gsem)

    def gather_wait(j):
        n = sizes[j]
        pltpu.make_async_copy(w_v.at[idx_v.at[pl.ds(_OFFS[j], n)]],
                              bufs[j % _NBUF].at[pl.ds(0, n)], gsem).wait()

    def scatter(j, wait):
        n = sizes[j]
        src = bufs[j % _NBUF].at[pl.ds(0, n)]
        dst = out_hbm.at[pl.ds(pl.multiple_of(base + _OFFS[j], 8), n)]
        if wait:
            pltpu.make_async_copy(src, dst, ssem).wait()
        else:
            pltpu.async_copy(src, dst, ssem)

    for j in range(_LAG):
        gather(j)
    for j in range(nch):
        gather_wait(j)
        scatter(j, wait=False)
        if j >= _LAG:
            scatter(j - _LAG, wait=True)
        if j + _LAG < nch:
            gather(j + _LAG)
    for j in range(nch - _LAG, nch):
        scatter(j, wait=True)


def _body(z_hbm, w_hbm, out_hbm, idx_v, w_v, b0, b1, b2, b3, b4, b5,
          gsem, ssem):
    bufs = (b0, b1, b2, b3, b4, b5)
    wid = lax.axis_index("s") * _INFO.num_cores + lax.axis_index("c")
    base = pl.multiple_of(wid * _SLAB, 8)

    @pl.when(lax.axis_index("s") == 0)
    def _():
        pltpu.sync_copy(w_hbm, w_v)

    plsc.subcore_barrier()

    @pl.when(wid < _NW - 1)
    def _():
        pltpu.sync_copy(z_hbm.at[pl.ds(base, _SLAB)], idx_v)
        _pipeline(base, _SIZES_MAIN, idx_v, w_v, out_hbm, bufs, gsem, ssem)

    @pl.when(wid == _NW - 1)
    def _():
        pltpu.sync_copy(z_hbm.at[pl.ds(base, _SLAB_LAST)],
                        idx_v.at[pl.ds(0, _SLAB_LAST)])
        _pipeline(base, _SIZES_LAST, idx_v, w_v, out_hbm, bufs, gsem, ssem)


@jax.jit
def kernel(Z, W):
    w95 = jnp.concatenate([jnp.zeros((1, _EMB), jnp.float32), W])
    mesh = plsc.VectorSubcoreMesh(core_axis_name="c", subcore_axis_name="s")
    return pl.kernel(
        _body,
        out_type=jax.ShapeDtypeStruct((_N_ATOMS, _EMB), jnp.float32),
        mesh=mesh,
        scratch_types=[
            pltpu.VMEM((_SLAB,), jnp.int32),
            pltpu.VMEM_SHARED((_N_ELEM + 1, _EMB), jnp.float32),
            pltpu.VMEM((_CHUNK, _EMB), jnp.float32),
            pltpu.VMEM((_CHUNK, _EMB), jnp.float32),
            pltpu.VMEM((_CHUNK, _EMB), jnp.float32),
            pltpu.VMEM((_CHUNK, _EMB), jnp.float32),
            pltpu.VMEM((_CHUNK, _EMB), jnp.float32),
            pltpu.VMEM((_CHUNK, _EMB), jnp.float32),
            pltpu.SemaphoreType.DMA,
            pltpu.SemaphoreType.DMA,
        ],
    )(Z, w95)
